# baseline (speedup 1.0000x reference)
import functools

import jax
import jax.numpy as jnp
from jax import lax
from jax.experimental import pallas as pl
from jax.experimental.pallas import tpu as pltpu
from jax.experimental.pallas import tpu_sc as plsc

_B, _L, _D = 4096, 200, 64
_NC, _NS = 2, 16
_NW = _NC * _NS
_BPW = _B // _NW
_NBUF = 4


def _gather_sc(idx, table):
  mesh = plsc.VectorSubcoreMesh(core_axis_name="c", subcore_axis_name="s")

  @functools.partial(
      pl.kernel,
      mesh=mesh,
      compiler_params=pltpu.CompilerParams(use_tc_tiling_on_sc=False),
      out_type=jax.ShapeDtypeStruct((_B, _L, _D), jnp.float32),
      scratch_types=[
          pltpu.VMEM((_BPW, _L), jnp.int32),
          pltpu.VMEM((_NBUF, _L, _D), jnp.float32),
          pltpu.SemaphoreType.DMA((_NBUF,)),
          pltpu.SemaphoreType.DMA((_NBUF,)),
      ],
  )
  def body(idx_hbm, table_hbm, out_hbm, idx_v, rows_v, gsem, ssem):
    wid = lax.axis_index("s") * _NC + lax.axis_index("c")
    row0 = wid * _BPW
    pltpu.sync_copy(idx_hbm.at[pl.ds(row0, _BPW)], idx_v)

    def gather(j, b):
      return pltpu.make_async_copy(
          table_hbm.at[idx_v.at[j]], rows_v.at[b], gsem.at[b])

    def store(j, b):
      return pltpu.make_async_copy(
          rows_v.at[b], out_hbm.at[row0 + j], ssem.at[b])

    for b in range(_NBUF):
      gather(b, b).start()

    def group(i, carry):
      j0 = i * _NBUF
      for b in range(_NBUF):
        gather(j0 + b, b).wait()
        store(j0 + b, b).start()
      for b in range(_NBUF):
        store(j0 + b, b).wait()
        gather(j0 + b + _NBUF, b).start()
      return carry

    lax.fori_loop(0, _BPW // _NBUF - 1, group, 0)

    j0 = _BPW - _NBUF
    for b in range(_NBUF):
      gather(j0 + b, b).wait()
      store(j0 + b, b).start()
    for b in range(_NBUF):
      store(j0 + b, b).wait()

  return body(idx, table)


def kernel(input_ids, token_types, word_embeddings):
  del token_types
  return _gather_sc(input_ids, word_embeddings)


# speedup vs baseline: 1.3308x; 1.3308x over previous
import functools

import jax
import jax.numpy as jnp
from jax import lax
from jax.experimental import pallas as pl
from jax.experimental.pallas import tpu as pltpu
from jax.experimental.pallas import tpu_sc as plsc

_B, _L, _D = 4096, 200, 64
_NC, _NS = 2, 16
_NW = _NC * _NS
_BPW = _B // _NW
_NBUF = 4


def _gather_sc(idx, table):
  mesh = plsc.VectorSubcoreMesh(core_axis_name="c", subcore_axis_name="s")

  @functools.partial(
      pl.kernel,
      mesh=mesh,
      compiler_params=pltpu.CompilerParams(use_tc_tiling_on_sc=False),
      out_type=jax.ShapeDtypeStruct((_B, _L, 128), jnp.float32),
      scratch_types=(
          [pltpu.VMEM((_BPW, _L), jnp.int32),
           pltpu.VMEM((_NBUF, _L, _D), jnp.float32)]
          + [pltpu.SemaphoreType.DMA] * (2 * _NBUF)
      ),
  )
  def body(idx_hbm, table_hbm, out_hbm, idx_v, rows_v, *sems):
    gsems, ssems = sems[:_NBUF], sems[_NBUF:]
    wid = lax.axis_index("s") * _NC + lax.axis_index("c")
    row0 = wid * _BPW
    pltpu.sync_copy(idx_hbm.at[pl.ds(row0, _BPW)], idx_v)

    def gather(j, b):
      return pltpu.make_async_copy(
          table_hbm.at[idx_v.at[j]], rows_v.at[b], gsems[b])

    def store(j, b):
      return pltpu.make_async_copy(
          rows_v.at[b], out_hbm.at[row0 + j, :, pl.ds(0, _D)], ssems[b])

    for b in range(_NBUF):
      gather(b, b).start()

    def group(i, carry):
      j0 = i * _NBUF
      for b in range(_NBUF):
        gather(j0 + b, b).wait()
        store(j0 + b, b).start()
      for b in range(_NBUF):
        store(j0 + b, b).wait()
        gather(j0 + b + _NBUF, b).start()
      return carry

    lax.fori_loop(0, _BPW // _NBUF - 1, group, 0)

    j0 = _BPW - _NBUF
    for b in range(_NBUF):
      gather(j0 + b, b).wait()
      store(j0 + b, b).start()
    for b in range(_NBUF):
      store(j0 + b, b).wait()

  return body(idx, table)


def kernel(input_ids, token_types, word_embeddings):
  del token_types
  out_pad = _gather_sc(input_ids, word_embeddings)
  return out_pad[:, :, :_D]
